# baseline (speedup 1.0000x reference)
import functools

import jax
import jax.numpy as jnp
from jax import lax
from jax.experimental import pallas as pl
from jax.experimental.pallas import tpu as pltpu
from jax.experimental.pallas import tpu_sc as plsc

_S = 256
_H = 128
_NC = 10
_NCORES = 2
_NSUB = 16
_NW = _NCORES * _NSUB


def _reduce_chunk(buf):
    zero = jnp.zeros((16,), jnp.float32)

    def row_body(r, accs):
        al, ar = accs
        for k in range(8):
            al = al + buf[r, pl.ds(k * 16, 16)]
            ar = ar + buf[r, pl.ds(_H + k * 16, 16)]
        return al, ar

    return lax.fori_loop(0, _H, row_body, (zero, zero))


def _sc_body(imgs_per_w, chunk0, x_hbm, out_hbm, buf0, buf1, outv, sem0, sem1):
    wid = lax.axis_index("s") * _NCORES + lax.axis_index("c")
    chunks_per_w = 2 * imgs_per_w
    obase = wid * imgs_per_w
    base = chunk0 + wid * chunks_per_w

    pltpu.make_async_copy(x_hbm.at[base], buf0, sem0).start()
    pltpu.make_async_copy(x_hbm.at[base + 1], buf1, sem1).start()

    lanes = lax.iota(jnp.int32, 16)
    omask = lanes < 4

    def img_body(i, carry):
        c0 = base + 2 * i
        pltpu.make_async_copy(x_hbm.at[c0], buf0, sem0).wait()
        al0, ar0 = _reduce_chunk(buf0)

        @pl.when(i < imgs_per_w - 1)
        def _():
            pltpu.make_async_copy(x_hbm.at[c0 + 2], buf0, sem0).start()

        pltpu.make_async_copy(x_hbm.at[c0 + 1], buf1, sem1).wait()
        al1, ar1 = _reduce_chunk(buf1)

        @pl.when(i < imgs_per_w - 1)
        def _():
            pltpu.make_async_copy(x_hbm.at[c0 + 3], buf1, sem1).start()

        q0 = jnp.sum(al0)
        q3 = jnp.sum(ar0)
        q1 = jnp.sum(al1)
        q2 = jnp.sum(ar1)
        vec = jnp.where(lanes == 0, q0,
                        jnp.where(lanes == 1, q1,
                                  jnp.where(lanes == 2, q2, q3)))
        row = jnp.full((16,), i, jnp.int32)
        plsc.store_scatter(outv, [row, lanes], vec, mask=omask)
        return 0

    lax.fori_loop(0, imgs_per_w, img_body, 0)
    pltpu.sync_copy(outv, out_hbm.at[pl.ds(obase, imgs_per_w), :])


def _sc_part(x, k_sc):
    n = x.shape[0]
    imgs_per_w = k_sc // _NW
    x2 = x.reshape(2 * n, _H, _S)
    mesh = plsc.VectorSubcoreMesh(core_axis_name="c", subcore_axis_name="s")
    return pl.kernel(
        functools.partial(_sc_body, imgs_per_w, 2 * (n - k_sc)),
        out_type=jax.ShapeDtypeStruct((k_sc, 4), jnp.float32),
        mesh=mesh,
        compiler_params=pltpu.CompilerParams(needs_layout_passes=False),
        scratch_types=[
            pltpu.VMEM((_H, _S), jnp.float32),
            pltpu.VMEM((_H, _S), jnp.float32),
            pltpu.VMEM((imgs_per_w, 4), jnp.float32),
            pltpu.SemaphoreType.DMA,
            pltpu.SemaphoreType.DMA,
        ],
    )(x2)


def _fold_and_apply(tl, bl, br, tr, v_ref, g_ref, b_ref, w_ref, fb_ref):
    nb = tl.shape[0]
    v3 = v_ref[...]
    vsq = jnp.sum(v3 * v3, axis=2, keepdims=True)
    w3 = g_ref[...] * v3 * lax.rsqrt(vsq)
    b = b_ref[...]
    wrow = jnp.concatenate([w3[i] for i in range(4)], axis=1)
    brow = jnp.concatenate([b[i][None, :] for i in range(4)], axis=1)
    aggrep = jnp.concatenate(
        [jnp.broadcast_to(t[:, None], (nb, 4)) for t in (tl, bl, br, tr)],
        axis=1)
    y16 = aggrep * wrow + brow
    return (lax.dot_general(y16, w_ref[...], (((1,), (1,)), ((), ())),
                            preferred_element_type=jnp.float32)
            + fb_ref[...])


def _tc_body(x_ref, v_ref, g_ref, b_ref, w_ref, fb_ref, out_ref):
    xb = x_ref[...]
    tl = jnp.sum(xb[:, :_H, :_H], axis=(1, 2))
    bl = jnp.sum(xb[:, _H:, :_H], axis=(1, 2))
    br = jnp.sum(xb[:, _H:, _H:], axis=(1, 2))
    tr = jnp.sum(xb[:, :_H, _H:], axis=(1, 2))
    out_ref[...] = _fold_and_apply(tl, bl, br, tr,
                                   v_ref, g_ref, b_ref, w_ref, fb_ref)


def _aff_body(agg_ref, v_ref, g_ref, b_ref, w_ref, fb_ref, out_ref):
    agg = agg_ref[...]
    out_ref[...] = _fold_and_apply(agg[:, 0], agg[:, 1], agg[:, 2], agg[:, 3],
                                   v_ref, g_ref, b_ref, w_ref, fb_ref)


def _aff_part(agg, fgl_v, fgl_g, fgl_b, fc_w, fc_b):
    k = agg.shape[0]
    return pl.pallas_call(
        _aff_body,
        grid=(1,),
        in_specs=[
            pl.BlockSpec((k, 4), lambda i: (0, 0)),
            pl.BlockSpec((4, 1, 4), lambda i: (0, 0, 0)),
            pl.BlockSpec((4, 1, 1), lambda i: (0, 0, 0)),
            pl.BlockSpec((4, 4), lambda i: (0, 0)),
            pl.BlockSpec((_NC, 16), lambda i: (0, 0)),
            pl.BlockSpec((1, _NC), lambda i: (0, 0)),
        ],
        out_specs=pl.BlockSpec((k, _NC), lambda i: (0, 0)),
        out_shape=jax.ShapeDtypeStruct((k, _NC), jnp.float32),
    )(agg, fgl_v, fgl_g, fgl_b, fc_w, fc_b.reshape(1, _NC))


_BB = 32


def _tc_part(x, k_sc, fgl_v, fgl_g, fgl_b, fc_w, fc_b):
    n = x.shape[0]
    return pl.pallas_call(
        _tc_body,
        grid=((n - k_sc) // _BB,),
        in_specs=[
            pl.BlockSpec((_BB, _S, _S), lambda i: (i, 0, 0)),
            pl.BlockSpec((4, 1, 4), lambda i: (0, 0, 0)),
            pl.BlockSpec((4, 1, 1), lambda i: (0, 0, 0)),
            pl.BlockSpec((4, 4), lambda i: (0, 0)),
            pl.BlockSpec((_NC, 16), lambda i: (0, 0)),
            pl.BlockSpec((1, _NC), lambda i: (0, 0)),
        ],
        out_specs=pl.BlockSpec((_BB, _NC), lambda i: (i, 0)),
        out_shape=jax.ShapeDtypeStruct((n - k_sc, _NC), jnp.float32),
    )(x, fgl_v, fgl_g, fgl_b, fc_w, fc_b.reshape(1, _NC))


_K_SC = 256


def kernel(x, fgl_v, fgl_g, fgl_b, fc_w, fc_b):
    out_tc = _tc_part(x, _K_SC, fgl_v, fgl_g, fgl_b, fc_w, fc_b)
    agg_sc = _sc_part(x, _K_SC)
    out_sc = _aff_part(agg_sc, fgl_v, fgl_g, fgl_b, fc_w, fc_b)
    return jnp.concatenate([out_tc, out_sc], axis=0)


# speedup vs baseline: 1.0178x; 1.0178x over previous
import functools

import jax
import jax.numpy as jnp
from jax import lax
from jax.experimental import pallas as pl
from jax.experimental.pallas import tpu as pltpu
from jax.experimental.pallas import tpu_sc as plsc

_S = 256
_H = 128
_NC = 10
_NCORES = 2
_NSUB = 16
_NW = _NCORES * _NSUB


def _reduce_chunk(buf):
    zero = jnp.zeros((16,), jnp.float32)

    def row_body(r, accs):
        al, ar = accs
        for k in range(8):
            al = al + buf[r, pl.ds(k * 16, 16)]
            ar = ar + buf[r, pl.ds(_H + k * 16, 16)]
        return al, ar

    return lax.fori_loop(0, _H, row_body, (zero, zero))


def _sc_body(imgs_per_w, chunk0, x_hbm, out_hbm, buf0, buf1, outv, sem0, sem1):
    wid = lax.axis_index("s") * _NCORES + lax.axis_index("c")
    chunks_per_w = 2 * imgs_per_w
    obase = wid * imgs_per_w
    base = chunk0 + wid * chunks_per_w

    pltpu.make_async_copy(x_hbm.at[base], buf0, sem0).start()
    pltpu.make_async_copy(x_hbm.at[base + 1], buf1, sem1).start()

    lanes = lax.iota(jnp.int32, 16)
    omask = lanes < 4

    def img_body(i, carry):
        c0 = base + 2 * i
        pltpu.make_async_copy(x_hbm.at[c0], buf0, sem0).wait()
        al0, ar0 = _reduce_chunk(buf0)

        @pl.when(i < imgs_per_w - 1)
        def _():
            pltpu.make_async_copy(x_hbm.at[c0 + 2], buf0, sem0).start()

        pltpu.make_async_copy(x_hbm.at[c0 + 1], buf1, sem1).wait()
        al1, ar1 = _reduce_chunk(buf1)

        @pl.when(i < imgs_per_w - 1)
        def _():
            pltpu.make_async_copy(x_hbm.at[c0 + 3], buf1, sem1).start()

        q0 = jnp.sum(al0)
        q3 = jnp.sum(ar0)
        q1 = jnp.sum(al1)
        q2 = jnp.sum(ar1)
        vec = jnp.where(lanes == 0, q0,
                        jnp.where(lanes == 1, q1,
                                  jnp.where(lanes == 2, q2, q3)))
        row = jnp.full((16,), i, jnp.int32)
        plsc.store_scatter(outv, [row, lanes], vec, mask=omask)
        return 0

    lax.fori_loop(0, imgs_per_w, img_body, 0)
    pltpu.sync_copy(outv, out_hbm.at[pl.ds(obase, imgs_per_w), :])


def _sc_part(x, k_sc):
    n = x.shape[0]
    imgs_per_w = k_sc // _NW
    x2 = x.reshape(2 * n, _H, _S)
    mesh = plsc.VectorSubcoreMesh(core_axis_name="c", subcore_axis_name="s")
    return pl.kernel(
        functools.partial(_sc_body, imgs_per_w, 2 * (n - k_sc)),
        out_type=jax.ShapeDtypeStruct((k_sc, 4), jnp.float32),
        mesh=mesh,
        compiler_params=pltpu.CompilerParams(needs_layout_passes=False),
        scratch_types=[
            pltpu.VMEM((_H, _S), jnp.float32),
            pltpu.VMEM((_H, _S), jnp.float32),
            pltpu.VMEM((imgs_per_w, 4), jnp.float32),
            pltpu.SemaphoreType.DMA,
            pltpu.SemaphoreType.DMA,
        ],
    )(x2)


def _fold_and_apply(tl, bl, br, tr, v_ref, g_ref, b_ref, w_ref, fb_ref):
    nb = tl.shape[0]
    v3 = v_ref[...]
    vsq = jnp.sum(v3 * v3, axis=2, keepdims=True)
    w3 = g_ref[...] * v3 * lax.rsqrt(vsq)
    b = b_ref[...]
    wrow = jnp.concatenate([w3[i] for i in range(4)], axis=1)
    brow = jnp.concatenate([b[i][None, :] for i in range(4)], axis=1)
    aggrep = jnp.concatenate(
        [jnp.broadcast_to(t[:, None], (nb, 4)) for t in (tl, bl, br, tr)],
        axis=1)
    y16 = aggrep * wrow + brow
    return (lax.dot_general(y16, w_ref[...], (((1,), (1,)), ((), ())),
                            preferred_element_type=jnp.float32)
            + fb_ref[...])


def _tc_body(x_ref, v_ref, g_ref, b_ref, w_ref, fb_ref, out_ref):
    xb = x_ref[...]
    tl = jnp.sum(xb[:, :_H, :_H], axis=(1, 2))
    bl = jnp.sum(xb[:, _H:, :_H], axis=(1, 2))
    br = jnp.sum(xb[:, _H:, _H:], axis=(1, 2))
    tr = jnp.sum(xb[:, :_H, _H:], axis=(1, 2))
    out_ref[...] = _fold_and_apply(tl, bl, br, tr,
                                   v_ref, g_ref, b_ref, w_ref, fb_ref)


def _aff_body(agg_ref, v_ref, g_ref, b_ref, w_ref, fb_ref, out_ref):
    agg = agg_ref[...]
    out_ref[...] = _fold_and_apply(agg[:, 0], agg[:, 1], agg[:, 2], agg[:, 3],
                                   v_ref, g_ref, b_ref, w_ref, fb_ref)


def _aff_part(agg, fgl_v, fgl_g, fgl_b, fc_w, fc_b):
    k = agg.shape[0]
    return pl.pallas_call(
        _aff_body,
        grid=(1,),
        in_specs=[
            pl.BlockSpec((k, 4), lambda i: (0, 0)),
            pl.BlockSpec((4, 1, 4), lambda i: (0, 0, 0)),
            pl.BlockSpec((4, 1, 1), lambda i: (0, 0, 0)),
            pl.BlockSpec((4, 4), lambda i: (0, 0)),
            pl.BlockSpec((_NC, 16), lambda i: (0, 0)),
            pl.BlockSpec((1, _NC), lambda i: (0, 0)),
        ],
        out_specs=pl.BlockSpec((k, _NC), lambda i: (0, 0)),
        out_shape=jax.ShapeDtypeStruct((k, _NC), jnp.float32),
    )(agg, fgl_v, fgl_g, fgl_b, fc_w, fc_b.reshape(1, _NC))


_BB = 32


def _tc_part(x, k_sc, fgl_v, fgl_g, fgl_b, fc_w, fc_b):
    n = x.shape[0]
    return pl.pallas_call(
        _tc_body,
        grid=((n - k_sc) // _BB,),
        in_specs=[
            pl.BlockSpec((_BB, _S, _S), lambda i: (i, 0, 0)),
            pl.BlockSpec((4, 1, 4), lambda i: (0, 0, 0)),
            pl.BlockSpec((4, 1, 1), lambda i: (0, 0, 0)),
            pl.BlockSpec((4, 4), lambda i: (0, 0)),
            pl.BlockSpec((_NC, 16), lambda i: (0, 0)),
            pl.BlockSpec((1, _NC), lambda i: (0, 0)),
        ],
        out_specs=pl.BlockSpec((_BB, _NC), lambda i: (i, 0)),
        out_shape=jax.ShapeDtypeStruct((n - k_sc, _NC), jnp.float32),
    )(x, fgl_v, fgl_g, fgl_b, fc_w, fc_b.reshape(1, _NC))


_K_SC = 64


def kernel(x, fgl_v, fgl_g, fgl_b, fc_w, fc_b):
    out_tc = _tc_part(x, _K_SC, fgl_v, fgl_g, fgl_b, fc_w, fc_b)
    agg_sc = _sc_part(x, _K_SC)
    out_sc = _aff_part(agg_sc, fgl_v, fgl_g, fgl_b, fc_w, fc_b)
    return jnp.concatenate([out_tc, out_sc], axis=0)
